# baseline (speedup 1.0000x reference)
import functools
import jax
import jax.numpy as jnp
from jax import lax
from jax.experimental import pallas as pl
from jax.experimental.pallas import tpu as pltpu
from jax.experimental.pallas import tpu_sc as plsc

NUM_CLASSES = 100000
DIM = 64
BATCH = 16384
NC = 2
NS = 16
NW = NC * NS
FEATS_PER_W = DIM // NW
LANES = 16
STRIP = 4096
NSTRIP = BATCH // STRIP


def _sc_body(y_hbm, dft_hbm, ctt_hbm, out_hbm, row_v, y_v, df_v, acc_v,
             rsem, ysem, dsem):
    wid = lax.axis_index("s") * NC + lax.axis_index("c")

    yc = pltpu.async_copy(y_hbm, y_v, ysem)

    def strip_copy(c, s, b):
        return pltpu.async_copy(dft_hbm.at[c, pl.ds(s * STRIP, STRIP)],
                                df_v.at[pl.ds(b * STRIP, STRIP)], dsem)

    def strip_compute(s, b, acc):
        ybase = s * STRIP
        dbase = b * STRIP

        @plsc.parallel_loop(0, STRIP, 8 * LANES, unroll=2, carry=acc)
        def loop(pos, accs):
            out = []
            for u in range(8):
                p = pos + u * LANES
                yv = y_v[pl.ds(ybase + p, LANES)]
                g = plsc.load_gather(row_v, [yv])
                d = df_v[pl.ds(dbase + p, LANES)] - g
                out.append(accs[u] + d * d)
            return tuple(out)

        return loop

    acc = tuple(jnp.zeros((LANES,), jnp.float32) for _ in range(8))
    first = True
    for f in range(FEATS_PER_W):
        c = wid + NW * f
        rc = pltpu.async_copy(ctt_hbm.at[c], row_v, rsem)
        cp = strip_copy(c, 0, 0)
        rc.wait()
        if first:
            yc.wait()
            first = False
        for s in range(NSTRIP):
            b = s % 2
            nxt = None
            if s + 1 < NSTRIP:
                nxt = strip_copy(c, s + 1, 1 - b)
            cp.wait()
            acc = strip_compute(s, b, acc)
            cp = nxt

    acc_v[...] = (((acc[0] + acc[1]) + (acc[2] + acc[3]))
                  + ((acc[4] + acc[5]) + (acc[6] + acc[7])))
    pltpu.sync_copy(
        acc_v, out_hbm.at[pl.ds(pl.multiple_of(wid * LANES, LANES), LANES)])


_sc_call = pl.kernel(
    _sc_body,
    out_type=jax.ShapeDtypeStruct((NW * LANES,), jnp.float32),
    mesh=plsc.VectorSubcoreMesh(core_axis_name="c", subcore_axis_name="s"),
    compiler_params=pltpu.CompilerParams(use_tc_tiling_on_sc=True,
                                         needs_layout_passes=False,
                                         disable_bounds_checks=True,
                                         disable_semaphore_checks=True),
    scratch_types=[
        pltpu.VMEM((NUM_CLASSES,), jnp.float32),
        pltpu.VMEM((BATCH,), jnp.int32),
        pltpu.VMEM((2 * STRIP,), jnp.float32),
        pltpu.VMEM((LANES,), jnp.float32),
        pltpu.SemaphoreType.DMA,
        pltpu.SemaphoreType.DMA,
        pltpu.SemaphoreType.DMA,
    ],
)


@jax.jit
def kernel(y, deep_feat, centers):
    partials = _sc_call(y.astype(jnp.int32), deep_feat.T, centers.T)
    return 0.5 * jnp.sqrt(jnp.sum(partials)) / BATCH


# speedup vs baseline: 1.0038x; 1.0038x over previous
import functools
import jax
import jax.numpy as jnp
from jax import lax
from jax.experimental import pallas as pl
from jax.experimental.pallas import tpu as pltpu
from jax.experimental.pallas import tpu_sc as plsc

NUM_CLASSES = 100000
DIM = 64
BATCH = 16384
NC = 2
NS = 16
NW = NC * NS
FEATS_PER_W = DIM // NW
LANES = 16
STRIP = 4096
NSTRIP = BATCH // STRIP


def _sc_body(y_hbm, dft_hbm, ctt_hbm, out_hbm, row_v, y_v, df_v, acc_v,
             rsem, ysem, dsem):
    wid = lax.axis_index("s") * NC + lax.axis_index("c")

    yc = pltpu.async_copy(y_hbm, y_v, ysem)

    def strip_copy(c, s, b):
        return pltpu.async_copy(dft_hbm.at[c, pl.ds(s * STRIP, STRIP)],
                                df_v.at[pl.ds(b * STRIP, STRIP)], dsem)

    def strip_compute(s, b, acc):
        ybase = s * STRIP
        dbase = b * STRIP

        @plsc.parallel_loop(0, STRIP, 4 * LANES, unroll=4, carry=acc)
        def loop(pos, accs):
            out = []
            for u in range(4):
                p = pos + u * LANES
                yv = y_v[pl.ds(ybase + p, LANES)]
                g = plsc.load_gather(row_v, [yv])
                d = df_v[pl.ds(dbase + p, LANES)] - g
                out.append(accs[u] + d * d)
            return tuple(out)

        return loop

    acc = tuple(jnp.zeros((LANES,), jnp.float32) for _ in range(4))
    first = True
    for f in range(FEATS_PER_W):
        c = wid + NW * f
        rc = pltpu.async_copy(ctt_hbm.at[c], row_v, rsem)
        cp = strip_copy(c, 0, 0)
        rc.wait()
        if first:
            yc.wait()
            first = False
        for s in range(NSTRIP):
            b = s % 2
            nxt = None
            if s + 1 < NSTRIP:
                nxt = strip_copy(c, s + 1, 1 - b)
            cp.wait()
            acc = strip_compute(s, b, acc)
            cp = nxt

    acc_v[...] = (acc[0] + acc[1]) + (acc[2] + acc[3])
    pltpu.sync_copy(
        acc_v, out_hbm.at[pl.ds(pl.multiple_of(wid * LANES, LANES), LANES)])


_sc_call = pl.kernel(
    _sc_body,
    out_type=jax.ShapeDtypeStruct((NW * LANES,), jnp.float32),
    mesh=plsc.VectorSubcoreMesh(core_axis_name="c", subcore_axis_name="s"),
    compiler_params=pltpu.CompilerParams(use_tc_tiling_on_sc=True,
                                         needs_layout_passes=False,
                                         disable_bounds_checks=True,
                                         disable_semaphore_checks=True),
    scratch_types=[
        pltpu.VMEM((NUM_CLASSES,), jnp.float32),
        pltpu.VMEM((BATCH,), jnp.int32),
        pltpu.VMEM((2 * STRIP,), jnp.float32),
        pltpu.VMEM((LANES,), jnp.float32),
        pltpu.SemaphoreType.DMA,
        pltpu.SemaphoreType.DMA,
        pltpu.SemaphoreType.DMA,
    ],
)


@jax.jit
def kernel(y, deep_feat, centers):
    partials = _sc_call(y.astype(jnp.int32), deep_feat.T, centers.T)
    return 0.5 * jnp.sqrt(jnp.sum(partials)) / BATCH


# speedup vs baseline: 1.0231x; 1.0193x over previous
import functools
import jax
import jax.numpy as jnp
from jax import lax
from jax.experimental import pallas as pl
from jax.experimental.pallas import tpu as pltpu
from jax.experimental.pallas import tpu_sc as plsc

NUM_CLASSES = 100000
DIM = 64
BATCH = 16384
NC = 2
NS = 16
NW = NC * NS
FEATS_PER_W = DIM // NW
LANES = 16
STRIP = 4096
NSTRIP = BATCH // STRIP


def _sc_body(y_hbm, dft_hbm, ctt_hbm, out_hbm, row_v, y_v, df_v, acc_v,
             rsem, ysem, dsem):
    wid = lax.axis_index("s") * NC + lax.axis_index("c")

    yc = pltpu.async_copy(y_hbm, y_v, ysem)

    def strip_copy(c, s, b):
        return pltpu.async_copy(dft_hbm.at[c, pl.ds(s * STRIP, STRIP)],
                                df_v.at[pl.ds(b * STRIP, STRIP)], dsem)

    def strip_compute(s, b, acc):
        ybase = s * STRIP
        dbase = b * STRIP

        @plsc.parallel_loop(0, STRIP, 4 * LANES, unroll=2, carry=acc)
        def loop(pos, accs):
            out = []
            for u in range(4):
                p = pos + u * LANES
                yv = y_v[pl.ds(ybase + p, LANES)]
                g = plsc.load_gather(row_v, [yv])
                d = df_v[pl.ds(dbase + p, LANES)] - g
                out.append(accs[u] + d * d)
            return tuple(out)

        return loop

    acc = tuple(jnp.zeros((LANES,), jnp.float32) for _ in range(4))
    first = True
    for f in range(FEATS_PER_W):
        c = wid + NW * f
        rc = pltpu.async_copy(ctt_hbm.at[c], row_v, rsem)
        cp = strip_copy(c, 0, 0)
        rc.wait()
        if first:
            yc.wait()
            first = False
        for s in range(NSTRIP):
            b = s % 2
            nxt = None
            if s + 1 < NSTRIP:
                nxt = strip_copy(c, s + 1, 1 - b)
            cp.wait()
            acc = strip_compute(s, b, acc)
            cp = nxt

    acc_v[...] = (acc[0] + acc[1]) + (acc[2] + acc[3])
    pltpu.sync_copy(
        acc_v, out_hbm.at[pl.ds(pl.multiple_of(wid * LANES, LANES), LANES)])


_sc_call = pl.kernel(
    _sc_body,
    out_type=jax.ShapeDtypeStruct((NW * LANES,), jnp.float32),
    mesh=plsc.VectorSubcoreMesh(core_axis_name="c", subcore_axis_name="s"),
    compiler_params=pltpu.CompilerParams(use_tc_tiling_on_sc=True,
                                         needs_layout_passes=False,
                                         disable_bounds_checks=True,
                                         disable_semaphore_checks=True),
    scratch_types=[
        pltpu.VMEM((NUM_CLASSES,), jnp.float32),
        pltpu.VMEM((BATCH,), jnp.int32),
        pltpu.VMEM((2 * STRIP,), jnp.float32),
        pltpu.VMEM((LANES,), jnp.float32),
        pltpu.SemaphoreType.DMA,
        pltpu.SemaphoreType.DMA,
        pltpu.SemaphoreType.DMA,
    ],
)


@jax.jit
def kernel(y, deep_feat, centers):
    partials = _sc_call(y.astype(jnp.int32), deep_feat.T, centers.T)
    return 0.5 * jnp.sqrt(jnp.sum(partials)) / BATCH
